# baseline (speedup 1.0000x reference)
import functools

import jax
import jax.numpy as jnp
from jax import lax
from jax.experimental import pallas as pl
from jax.experimental.pallas import tpu as pltpu
from jax.experimental.pallas import tpu_sc as plsc

D = 64
S = 200
H = 128
C = 16
L = 16

CH = 16
G = 2
GS = G * S
R = 80
NGD = GS // R
GPC = CH // G


VB = 256
SLAB_W = VB * D
N_FULL = 1000000 // VB
PER_TILE = 122
TAIL_V = 1000000 - N_FULL * VB
SPAD = 17


def _transpose_sc(emb_t):
    info = plsc.get_sparse_core_info()
    nc = info.num_cores
    mesh = plsc.VectorSubcoreMesh(core_axis_name="c", subcore_axis_name="s")

    @functools.partial(
        pl.kernel,
        mesh=mesh,
        out_type=jax.ShapeDtypeStruct((1000000 * D,), jnp.float32),
        compiler_params=pltpu.CompilerParams(
            use_tc_tiling_on_sc=True, needs_layout_passes=False),
        scratch_types=[
            pltpu.VMEM((D, VB + SPAD), jnp.float32),
            pltpu.VMEM((D, VB + SPAD), jnp.float32),
            pltpu.VMEM((SLAB_W,), jnp.float32),
            pltpu.VMEM((SLAB_W,), jnp.float32),
            pltpu.VMEM((D, TAIL_V), jnp.float32),
            pltpu.SemaphoreType.DMA,
            pltpu.SemaphoreType.DMA,
            pltpu.SemaphoreType.DMA,
        ],
    )
    def tkern(embt_hbm, out_hbm, s0, s1, o0, o1, t0, isem0, isem1, osem):
        wid = lax.axis_index("s") * nc + lax.axis_index("c")
        base_slab = wid * PER_TILE
        iota16 = lax.iota(jnp.int32, 16)

        def fire_slab(slab, sbuf, sem):
            pltpu.async_copy(embt_hbm.at[:, pl.ds(slab * VB, VB)],
                             sbuf.at[:, pl.ds(0, VB)], sem)

        def drain_slab(sbuf, sem):
            pltpu.make_async_copy(
                embt_hbm.at[:, pl.ds(0, VB)],
                sbuf.at[:, pl.ds(0, VB)], sem).wait()

        def transpose_slab(sbuf, obuf, width):
            pass

        def drain_flush():
            pltpu.make_async_copy(
                o0, out_hbm.at[pl.ds(0, SLAB_W)], osem).wait()

        fire_slab(base_slab, s0, isem0)
        bufs = ((s0, isem0, o0), (s1, isem1, o1))

        def outer(k, carry):
            for par in range(2):
                sb, se, ob = bufs[par]
                nsb, nse, _ = bufs[1 - par]
                g = 2 * k + par
                slab = base_slab + g

                @pl.when(g + 1 < PER_TILE)
                def _():
                    fire_slab(slab + 1, nsb, nse)
                drain_slab(sb, se)
                transpose_slab(sb, ob, VB)

                @pl.when(g >= 2)
                def _():
                    drain_flush()
                pltpu.async_copy(
                    ob, out_hbm.at[pl.ds(slab * SLAB_W, SLAB_W)], osem)
            return carry

        lax.fori_loop(0, PER_TILE // 2, outer, 0)
        drain_flush()
        drain_flush()

        for w, slab in ((0, N_FULL - 2), (1, N_FULL - 1)):
            @pl.when(wid == w)
            def _():
                pltpu.sync_copy(embt_hbm.at[:, pl.ds(slab * VB, VB)],
                                s0.at[:, pl.ds(0, VB)])
                transpose_slab(s0, o0, VB)
                pltpu.sync_copy(o0, out_hbm.at[pl.ds(slab * SLAB_W, SLAB_W)])

        @pl.when(wid == 2)
        def _():
            pltpu.sync_copy(embt_hbm.at[:, pl.ds(N_FULL * VB, TAIL_V)], t0)
            transpose_slab(t0, o0, TAIL_V)
            pltpu.sync_copy(
                o0.at[pl.ds(0, TAIL_V * D)],
                out_hbm.at[pl.ds(N_FULL * VB * D, TAIL_V * D)])

    return tkern(emb_t)


def _transpose_tc(emb_t):
    nvoc = emb_t.shape[1]
    BW = 1024
    grid = (nvoc + BW - 1) // BW

    def body(x_ref, o_ref):
        x = x_ref[...]
        eye = jnp.eye(D, dtype=jnp.float32)
        t = jax.lax.dot_general(x, eye, (((0,), (0,)), ((), ())),
                                precision=jax.lax.Precision.HIGHEST,
                                preferred_element_type=jnp.float32)
        o_ref[...] = jnp.concatenate([t[0 : BW // 2], t[BW // 2 :]], axis=1)

    return pl.pallas_call(
        body,
        grid=(grid,),
        in_specs=[pl.BlockSpec((D, BW), lambda i: (0, i))],
        out_specs=pl.BlockSpec((BW // 2, 2 * D), lambda i: (i, 0)),
        out_shape=jax.ShapeDtypeStruct((nvoc // 2, 2 * D), jnp.float32),
    )(emb_t)


def _pool_sc(ids_flat, table, batch):
    info = plsc.get_sparse_core_info()
    nc, ns = info.num_cores, info.num_subcores
    nw = nc * ns
    b_per_w = batch // nw
    n_chunks = b_per_w // CH

    mesh = plsc.VectorSubcoreMesh(core_axis_name="c", subcore_axis_name="s")

    @functools.partial(
        pl.kernel,
        mesh=mesh,
        out_type=jax.ShapeDtypeStruct((batch, D), jnp.float32),
        compiler_params=pltpu.CompilerParams(use_tc_tiling_on_sc=False),
        scratch_types=[
            pltpu.VMEM((CH * S,), jnp.int32),
            pltpu.VMEM((CH * S,), jnp.int32),
            pltpu.VMEM((GS, D), jnp.float32),
            pltpu.VMEM((GS, D), jnp.float32),
            pltpu.VMEM((CH, D), jnp.float32),
            pltpu.VMEM((CH, D), jnp.float32),
            pltpu.SemaphoreType.DMA,
            pltpu.SemaphoreType.DMA,
            pltpu.SemaphoreType.DMA,
            pltpu.SemaphoreType.DMA,
        ],
    )
    def pool(ids_hbm, table_hbm, out_hbm, idx0, idx1, rb0, rb1, sm0, sm1,
             isem, gsem0, gsem1, osem):
        wid = lax.axis_index("s") * nc + lax.axis_index("c")
        base = wid * b_per_w
        idxs = (idx0, idx1)
        rbs = (rb0, rb1)
        gsems = (gsem0, gsem1)
        sms = (sm0, sm1)

        def fire(idxbuf, gi, rb, sem):
            for c in range(NGD):
                pltpu.async_copy(
                    table_hbm.at[idxbuf.at[pl.ds(gi * GS + c * R, R)]],
                    rb.at[pl.ds(c * R, R)],
                    sem,
                )

        def drain_rows(rb, sem):
            pltpu.make_async_copy(table_hbm.at[pl.ds(0, GS)], rb, sem).wait()

        def drain_idx(idxbuf):
            pltpu.make_async_copy(
                ids_hbm.at[pl.ds(0, CH * S)], idxbuf, isem).wait()

        def drain_out():
            pltpu.make_async_copy(
                sms[0], out_hbm.at[pl.ds(base, CH)], osem).wait()

        def reduce(rb, smbuf, e0):
            def body(j, accs):
                accs = list(accs)
                for u in range(8):
                    r = j * 8 + u
                    for d in range(4):
                        accs[d] = accs[d] + rb[r, pl.ds(d * L, L)]
                        accs[4 + d] = accs[4 + d] + rb[S + r, pl.ds(d * L, L)]
                return tuple(accs)

            z = jnp.zeros((L,), jnp.float32)
            accs = lax.fori_loop(0, S // 8, body, (z,) * 8)
            for d in range(4):
                smbuf[e0, pl.ds(d * L, L)] = accs[d]
                smbuf[e0 + 1, pl.ds(d * L, L)] = accs[4 + d]

        def cbody(ci, p):
            start = base + ci * CH
            for gi in range(GPC):
                par = gi % 2
                if gi < GPC - 1:
                    fire(idxs[p], gi + 1, rbs[1 - par], gsems[1 - par])
                else:
                    @pl.when(ci + 1 < n_chunks)
                    def _():
                        drain_idx(idxs[1 - p])
                        fire(idxs[1 - p], 0, rbs[0], gsems[0])
                drain_rows(rbs[par], gsems[par])
                reduce(rbs[par], sms[p], G * gi)

            @pl.when(ci > 0)
            def _():
                drain_out()
            pltpu.async_copy(sms[p], out_hbm.at[pl.ds(start, CH)], osem)

            @pl.when(ci + 2 < n_chunks)
            def _():
                pltpu.async_copy(
                    ids_hbm.at[pl.ds((start + 2 * CH) * S, CH * S)],
                    idxs[p], isem)

        pltpu.sync_copy(ids_hbm.at[pl.ds(base * S, CH * S)], idx0)
        pltpu.async_copy(
            ids_hbm.at[pl.ds((base + CH) * S, CH * S)], idx1, isem)
        fire(idx0, 0, rb0, gsem0)

        def outer(i, carry):
            cbody(2 * i, 0)
            cbody(2 * i + 1, 1)
            return carry

        lax.fori_loop(0, n_chunks // 2, outer, 0)
        drain_out()

    return pool(ids_flat, table)


def _mlp_tc(x_sums, w1, b1, w2, b2):
    batch = x_sums.shape[0]
    bt = 2048

    def mlp_body(x_ref, w1_ref, b1_ref, w2_ref, b2_ref, o_ref):
        xs = x_ref[...] * (1.0 / S)
        h = jnp.dot(xs, w1_ref[...], preferred_element_type=jnp.float32)
        h = jnp.maximum(h + b1_ref[...], 0.0)
        o_ref[...] = (
            jnp.dot(h, w2_ref[...], preferred_element_type=jnp.float32)
            + b2_ref[...]
        )

    return pl.pallas_call(
        mlp_body,
        grid=(batch // bt,),
        in_specs=[
            pl.BlockSpec((bt, D), lambda i: (i, 0)),
            pl.BlockSpec((D, H), lambda i: (0, 0)),
            pl.BlockSpec((1, H), lambda i: (0, 0)),
            pl.BlockSpec((H, C), lambda i: (0, 0)),
            pl.BlockSpec((1, C), lambda i: (0, 0)),
        ],
        out_specs=pl.BlockSpec((bt, C), lambda i: (i, 0)),
        out_shape=jax.ShapeDtypeStruct((batch, C), jnp.float32),
    )(x_sums, w1, b1.reshape(1, H), w2, b2.reshape(1, C))


def kernel(input_ids, emb, W1, b1, W2, b2):
    batch = input_ids.shape[0]
    ids = input_ids.reshape(-1).astype(jnp.int32)
    u = ids % 1024
    ids_flat = ids - u + jnp.where(u < 512, 2 * u, 2 * u - 1023)
    table_lin = _transpose_tc(emb.T).reshape(emb.shape)
    sums = _pool_sc(ids_flat, table_lin, batch)
    return _mlp_tc(sums, W1, b1, W2, b2)


# speedup vs baseline: 1.7367x; 1.7367x over previous
import functools

import jax
import jax.numpy as jnp
from jax import lax
from jax.experimental import pallas as pl
from jax.experimental.pallas import tpu as pltpu
from jax.experimental.pallas import tpu_sc as plsc

D = 64
S = 200
H = 128
C = 16
L = 16

CH = 16
G = 2
GS = G * S
R = 80
NGD = GS // R
GPC = CH // G


TBW = 4096
VB = 256
SLAB_W = VB * D
N_FULL = 1000000 // VB
PER_TILE = 122
TAIL_V = 1000000 - N_FULL * VB
SPAD = 17


def _transpose_sc(emb_t):
    info = plsc.get_sparse_core_info()
    nc = info.num_cores
    mesh = plsc.VectorSubcoreMesh(core_axis_name="c", subcore_axis_name="s")

    @functools.partial(
        pl.kernel,
        mesh=mesh,
        out_type=jax.ShapeDtypeStruct((1000000 * D,), jnp.float32),
        compiler_params=pltpu.CompilerParams(
            use_tc_tiling_on_sc=True, needs_layout_passes=False),
        scratch_types=[
            pltpu.VMEM((D, VB + SPAD), jnp.float32),
            pltpu.VMEM((D, VB + SPAD), jnp.float32),
            pltpu.VMEM((SLAB_W,), jnp.float32),
            pltpu.VMEM((SLAB_W,), jnp.float32),
            pltpu.VMEM((D, TAIL_V), jnp.float32),
            pltpu.SemaphoreType.DMA,
            pltpu.SemaphoreType.DMA,
            pltpu.SemaphoreType.DMA,
        ],
    )
    def tkern(embt_hbm, out_hbm, s0, s1, o0, o1, t0, isem0, isem1, osem):
        wid = lax.axis_index("s") * nc + lax.axis_index("c")
        base_slab = wid * PER_TILE
        iota16 = lax.iota(jnp.int32, 16)

        def fire_slab(slab, sbuf, sem):
            pltpu.async_copy(embt_hbm.at[:, pl.ds(slab * VB, VB)],
                             sbuf.at[:, pl.ds(0, VB)], sem)

        def drain_slab(sbuf, sem):
            pltpu.make_async_copy(
                embt_hbm.at[:, pl.ds(0, VB)],
                sbuf.at[:, pl.ds(0, VB)], sem).wait()

        def transpose_slab(sbuf, obuf, width):
            pass

        def drain_flush():
            pltpu.make_async_copy(
                o0, out_hbm.at[pl.ds(0, SLAB_W)], osem).wait()

        fire_slab(base_slab, s0, isem0)
        bufs = ((s0, isem0, o0), (s1, isem1, o1))

        def outer(k, carry):
            for par in range(2):
                sb, se, ob = bufs[par]
                nsb, nse, _ = bufs[1 - par]
                g = 2 * k + par
                slab = base_slab + g

                @pl.when(g + 1 < PER_TILE)
                def _():
                    fire_slab(slab + 1, nsb, nse)
                drain_slab(sb, se)
                transpose_slab(sb, ob, VB)

                @pl.when(g >= 2)
                def _():
                    drain_flush()
                pltpu.async_copy(
                    ob, out_hbm.at[pl.ds(slab * SLAB_W, SLAB_W)], osem)
            return carry

        lax.fori_loop(0, PER_TILE // 2, outer, 0)
        drain_flush()
        drain_flush()

        for w, slab in ((0, N_FULL - 2), (1, N_FULL - 1)):
            @pl.when(wid == w)
            def _():
                pltpu.sync_copy(embt_hbm.at[:, pl.ds(slab * VB, VB)],
                                s0.at[:, pl.ds(0, VB)])
                transpose_slab(s0, o0, VB)
                pltpu.sync_copy(o0, out_hbm.at[pl.ds(slab * SLAB_W, SLAB_W)])

        @pl.when(wid == 2)
        def _():
            pltpu.sync_copy(embt_hbm.at[:, pl.ds(N_FULL * VB, TAIL_V)], t0)
            transpose_slab(t0, o0, TAIL_V)
            pltpu.sync_copy(
                o0.at[pl.ds(0, TAIL_V * D)],
                out_hbm.at[pl.ds(N_FULL * VB * D, TAIL_V * D)])

    return tkern(emb_t)


def _transpose_tc(emb_t):
    nvoc = emb_t.shape[1]
    BW = TBW
    grid = (nvoc + BW - 1) // BW

    def body(x_ref, o_ref):
        t = x_ref[...].T
        o_ref[...] = jnp.concatenate([t[0 : BW // 2], t[BW // 2 :]], axis=1)

    return pl.pallas_call(
        body,
        grid=(grid,),
        in_specs=[pl.BlockSpec((D, BW), lambda i: (0, i))],
        out_specs=pl.BlockSpec((BW // 2, 2 * D), lambda i: (i, 0)),
        out_shape=jax.ShapeDtypeStruct((nvoc // 2, 2 * D), jnp.float32),
    )(emb_t)


def _pool_sc(ids_flat, table, batch):
    info = plsc.get_sparse_core_info()
    nc, ns = info.num_cores, info.num_subcores
    nw = nc * ns
    b_per_w = batch // nw
    n_chunks = b_per_w // CH

    mesh = plsc.VectorSubcoreMesh(core_axis_name="c", subcore_axis_name="s")

    @functools.partial(
        pl.kernel,
        mesh=mesh,
        out_type=jax.ShapeDtypeStruct((batch, D), jnp.float32),
        compiler_params=pltpu.CompilerParams(use_tc_tiling_on_sc=False),
        scratch_types=[
            pltpu.VMEM((CH * S,), jnp.int32),
            pltpu.VMEM((CH * S,), jnp.int32),
            pltpu.VMEM((GS, D), jnp.float32),
            pltpu.VMEM((GS, D), jnp.float32),
            pltpu.VMEM((CH, D), jnp.float32),
            pltpu.VMEM((CH, D), jnp.float32),
            pltpu.SemaphoreType.DMA,
            pltpu.SemaphoreType.DMA,
            pltpu.SemaphoreType.DMA,
            pltpu.SemaphoreType.DMA,
        ],
    )
    def pool(ids_hbm, table_hbm, out_hbm, idx0, idx1, rb0, rb1, sm0, sm1,
             isem, gsem0, gsem1, osem):
        wid = lax.axis_index("s") * nc + lax.axis_index("c")
        base = wid * b_per_w
        idxs = (idx0, idx1)
        rbs = (rb0, rb1)
        gsems = (gsem0, gsem1)
        sms = (sm0, sm1)

        def fire(idxbuf, gi, rb, sem):
            for c in range(NGD):
                pltpu.async_copy(
                    table_hbm.at[idxbuf.at[pl.ds(gi * GS + c * R, R)]],
                    rb.at[pl.ds(c * R, R)],
                    sem,
                )

        def drain_rows(rb, sem):
            pltpu.make_async_copy(table_hbm.at[pl.ds(0, GS)], rb, sem).wait()

        def drain_idx(idxbuf):
            pltpu.make_async_copy(
                ids_hbm.at[pl.ds(0, CH * S)], idxbuf, isem).wait()

        def drain_out():
            pltpu.make_async_copy(
                sms[0], out_hbm.at[pl.ds(base, CH)], osem).wait()

        def reduce(rb, smbuf, e0):
            def body(j, accs):
                accs = list(accs)
                for u in range(8):
                    r = j * 8 + u
                    for d in range(4):
                        accs[d] = accs[d] + rb[r, pl.ds(d * L, L)]
                        accs[4 + d] = accs[4 + d] + rb[S + r, pl.ds(d * L, L)]
                return tuple(accs)

            z = jnp.zeros((L,), jnp.float32)
            accs = lax.fori_loop(0, S // 8, body, (z,) * 8)
            for d in range(4):
                smbuf[e0, pl.ds(d * L, L)] = accs[d]
                smbuf[e0 + 1, pl.ds(d * L, L)] = accs[4 + d]

        def cbody(ci, p):
            start = base + ci * CH
            for gi in range(GPC):
                par = gi % 2
                if gi < GPC - 1:
                    fire(idxs[p], gi + 1, rbs[1 - par], gsems[1 - par])
                else:
                    @pl.when(ci + 1 < n_chunks)
                    def _():
                        drain_idx(idxs[1 - p])
                        fire(idxs[1 - p], 0, rbs[0], gsems[0])
                drain_rows(rbs[par], gsems[par])
                reduce(rbs[par], sms[p], G * gi)

            @pl.when(ci > 0)
            def _():
                drain_out()
            pltpu.async_copy(sms[p], out_hbm.at[pl.ds(start, CH)], osem)

            @pl.when(ci + 2 < n_chunks)
            def _():
                pltpu.async_copy(
                    ids_hbm.at[pl.ds((start + 2 * CH) * S, CH * S)],
                    idxs[p], isem)

        pltpu.sync_copy(ids_hbm.at[pl.ds(base * S, CH * S)], idx0)
        pltpu.async_copy(
            ids_hbm.at[pl.ds((base + CH) * S, CH * S)], idx1, isem)
        fire(idx0, 0, rb0, gsem0)

        def outer(i, carry):
            cbody(2 * i, 0)
            cbody(2 * i + 1, 1)
            return carry

        lax.fori_loop(0, n_chunks // 2, outer, 0)
        drain_out()

    return pool(ids_flat, table)


def _mlp_tc(x_sums, w1, b1, w2, b2):
    batch = x_sums.shape[0]
    bt = 2048

    def mlp_body(x_ref, w1_ref, b1_ref, w2_ref, b2_ref, o_ref):
        xs = x_ref[...] * (1.0 / S)
        h = jnp.dot(xs, w1_ref[...], preferred_element_type=jnp.float32)
        h = jnp.maximum(h + b1_ref[...], 0.0)
        o_ref[...] = (
            jnp.dot(h, w2_ref[...], preferred_element_type=jnp.float32)
            + b2_ref[...]
        )

    return pl.pallas_call(
        mlp_body,
        grid=(batch // bt,),
        in_specs=[
            pl.BlockSpec((bt, D), lambda i: (i, 0)),
            pl.BlockSpec((D, H), lambda i: (0, 0)),
            pl.BlockSpec((1, H), lambda i: (0, 0)),
            pl.BlockSpec((H, C), lambda i: (0, 0)),
            pl.BlockSpec((1, C), lambda i: (0, 0)),
        ],
        out_specs=pl.BlockSpec((bt, C), lambda i: (i, 0)),
        out_shape=jax.ShapeDtypeStruct((batch, C), jnp.float32),
    )(x_sums, w1, b1.reshape(1, H), w2, b2.reshape(1, C))


def kernel(input_ids, emb, W1, b1, W2, b2):
    batch = input_ids.shape[0]
    ids = input_ids.reshape(-1).astype(jnp.int32)
    u = ids % TBW
    ids_flat = ids - u + jnp.where(u < TBW // 2, 2 * u, 2 * u - (TBW - 1))
    table_lin = _transpose_tc(emb.T).reshape(emb.shape)
    sums = _pool_sc(ids_flat, table_lin, batch)
    return _mlp_tc(sums, W1, b1, W2, b2)


# speedup vs baseline: 1.7383x; 1.0009x over previous
import functools

import jax
import jax.numpy as jnp
from jax import lax
from jax.experimental import pallas as pl
from jax.experimental.pallas import tpu as pltpu
from jax.experimental.pallas import tpu_sc as plsc

D = 64
S = 200
H = 128
C = 16
L = 16

CH = 16
G = 2
GS = G * S
R = 80
NGD = GS // R
GPC = CH // G


TBW = 4096
VB = 256
SLAB_W = VB * D
N_FULL = 1000000 // VB
PER_TILE = 122
TAIL_V = 1000000 - N_FULL * VB
SPAD = 17


def _transpose_sc(emb_t):
    info = plsc.get_sparse_core_info()
    nc = info.num_cores
    mesh = plsc.VectorSubcoreMesh(core_axis_name="c", subcore_axis_name="s")

    @functools.partial(
        pl.kernel,
        mesh=mesh,
        out_type=jax.ShapeDtypeStruct((1000000 * D,), jnp.float32),
        compiler_params=pltpu.CompilerParams(
            use_tc_tiling_on_sc=True, needs_layout_passes=False),
        scratch_types=[
            pltpu.VMEM((D, VB + SPAD), jnp.float32),
            pltpu.VMEM((D, VB + SPAD), jnp.float32),
            pltpu.VMEM((SLAB_W,), jnp.float32),
            pltpu.VMEM((SLAB_W,), jnp.float32),
            pltpu.VMEM((D, TAIL_V), jnp.float32),
            pltpu.SemaphoreType.DMA,
            pltpu.SemaphoreType.DMA,
            pltpu.SemaphoreType.DMA,
        ],
    )
    def tkern(embt_hbm, out_hbm, s0, s1, o0, o1, t0, isem0, isem1, osem):
        wid = lax.axis_index("s") * nc + lax.axis_index("c")
        base_slab = wid * PER_TILE
        iota16 = lax.iota(jnp.int32, 16)

        def fire_slab(slab, sbuf, sem):
            pltpu.async_copy(embt_hbm.at[:, pl.ds(slab * VB, VB)],
                             sbuf.at[:, pl.ds(0, VB)], sem)

        def drain_slab(sbuf, sem):
            pltpu.make_async_copy(
                embt_hbm.at[:, pl.ds(0, VB)],
                sbuf.at[:, pl.ds(0, VB)], sem).wait()

        def transpose_slab(sbuf, obuf, width):
            pass

        def drain_flush():
            pltpu.make_async_copy(
                o0, out_hbm.at[pl.ds(0, SLAB_W)], osem).wait()

        fire_slab(base_slab, s0, isem0)
        bufs = ((s0, isem0, o0), (s1, isem1, o1))

        def outer(k, carry):
            for par in range(2):
                sb, se, ob = bufs[par]
                nsb, nse, _ = bufs[1 - par]
                g = 2 * k + par
                slab = base_slab + g

                @pl.when(g + 1 < PER_TILE)
                def _():
                    fire_slab(slab + 1, nsb, nse)
                drain_slab(sb, se)
                transpose_slab(sb, ob, VB)

                @pl.when(g >= 2)
                def _():
                    drain_flush()
                pltpu.async_copy(
                    ob, out_hbm.at[pl.ds(slab * SLAB_W, SLAB_W)], osem)
            return carry

        lax.fori_loop(0, PER_TILE // 2, outer, 0)
        drain_flush()
        drain_flush()

        for w, slab in ((0, N_FULL - 2), (1, N_FULL - 1)):
            @pl.when(wid == w)
            def _():
                pltpu.sync_copy(embt_hbm.at[:, pl.ds(slab * VB, VB)],
                                s0.at[:, pl.ds(0, VB)])
                transpose_slab(s0, o0, VB)
                pltpu.sync_copy(o0, out_hbm.at[pl.ds(slab * SLAB_W, SLAB_W)])

        @pl.when(wid == 2)
        def _():
            pltpu.sync_copy(embt_hbm.at[:, pl.ds(N_FULL * VB, TAIL_V)], t0)
            transpose_slab(t0, o0, TAIL_V)
            pltpu.sync_copy(
                o0.at[pl.ds(0, TAIL_V * D)],
                out_hbm.at[pl.ds(N_FULL * VB * D, TAIL_V * D)])

    return tkern(emb_t)


def _transpose_tc(emb_t):
    nvoc = emb_t.shape[1]
    BW = TBW
    grid = (nvoc + BW - 1) // BW
    nvoc_pad = grid * BW

    def body(x_ref, o_ref):
        t = x_ref[...].T
        o_ref[...] = jnp.concatenate([t[0 : BW // 2], t[BW // 2 :]], axis=1)

    return pl.pallas_call(
        body,
        grid=(grid,),
        in_specs=[pl.BlockSpec((D, BW), lambda i: (0, i))],
        out_specs=pl.BlockSpec((BW // 2, 2 * D), lambda i: (i, 0)),
        out_shape=jax.ShapeDtypeStruct((nvoc_pad // 2, 2 * D), jnp.float32),
    )(emb_t)


def _pool_sc(ids_flat, table, batch):
    info = plsc.get_sparse_core_info()
    nc, ns = info.num_cores, info.num_subcores
    nw = nc * ns
    b_per_w = batch // nw
    n_chunks = b_per_w // CH

    mesh = plsc.VectorSubcoreMesh(core_axis_name="c", subcore_axis_name="s")

    @functools.partial(
        pl.kernel,
        mesh=mesh,
        out_type=jax.ShapeDtypeStruct((batch, D), jnp.float32),
        compiler_params=pltpu.CompilerParams(use_tc_tiling_on_sc=False),
        scratch_types=[
            pltpu.VMEM((CH * S,), jnp.int32),
            pltpu.VMEM((CH * S,), jnp.int32),
            pltpu.VMEM((GS, D), jnp.float32),
            pltpu.VMEM((GS, D), jnp.float32),
            pltpu.VMEM((CH, D), jnp.float32),
            pltpu.VMEM((CH, D), jnp.float32),
            pltpu.SemaphoreType.DMA,
            pltpu.SemaphoreType.DMA,
            pltpu.SemaphoreType.DMA,
            pltpu.SemaphoreType.DMA,
        ],
    )
    def pool(ids_hbm, table_hbm, out_hbm, idx0, idx1, rb0, rb1, sm0, sm1,
             isem, gsem0, gsem1, osem):
        wid = lax.axis_index("s") * nc + lax.axis_index("c")
        base = wid * b_per_w
        idxs = (idx0, idx1)
        rbs = (rb0, rb1)
        gsems = (gsem0, gsem1)
        sms = (sm0, sm1)

        def fire(idxbuf, gi, rb, sem):
            for c in range(NGD):
                pltpu.async_copy(
                    table_hbm.at[idxbuf.at[pl.ds(gi * GS + c * R, R)]],
                    rb.at[pl.ds(c * R, R)],
                    sem,
                )

        def drain_rows(rb, sem):
            pltpu.make_async_copy(table_hbm.at[pl.ds(0, GS)], rb, sem).wait()

        def drain_idx(idxbuf):
            pltpu.make_async_copy(
                ids_hbm.at[pl.ds(0, CH * S)], idxbuf, isem).wait()

        def drain_out():
            pltpu.make_async_copy(
                sms[0], out_hbm.at[pl.ds(base, CH)], osem).wait()

        def reduce(rb, smbuf, e0):
            def body(j, accs):
                accs = list(accs)
                for u in range(8):
                    r = j * 8 + u
                    for d in range(4):
                        accs[d] = accs[d] + rb[r, pl.ds(d * L, L)]
                        accs[4 + d] = accs[4 + d] + rb[S + r, pl.ds(d * L, L)]
                return tuple(accs)

            z = jnp.zeros((L,), jnp.float32)
            accs = lax.fori_loop(0, S // 8, body, (z,) * 8)
            for d in range(4):
                smbuf[e0, pl.ds(d * L, L)] = accs[d]
                smbuf[e0 + 1, pl.ds(d * L, L)] = accs[4 + d]

        def cbody(ci, p):
            start = base + ci * CH
            for gi in range(GPC):
                par = gi % 2
                if gi < GPC - 1:
                    fire(idxs[p], gi + 1, rbs[1 - par], gsems[1 - par])
                else:
                    @pl.when(ci + 1 < n_chunks)
                    def _():
                        drain_idx(idxs[1 - p])
                        fire(idxs[1 - p], 0, rbs[0], gsems[0])
                drain_rows(rbs[par], gsems[par])
                reduce(rbs[par], sms[p], G * gi)

            @pl.when(ci > 0)
            def _():
                drain_out()
            pltpu.async_copy(sms[p], out_hbm.at[pl.ds(start, CH)], osem)

            @pl.when(ci + 2 < n_chunks)
            def _():
                pltpu.async_copy(
                    ids_hbm.at[pl.ds((start + 2 * CH) * S, CH * S)],
                    idxs[p], isem)

        pltpu.sync_copy(ids_hbm.at[pl.ds(base * S, CH * S)], idx0)
        pltpu.async_copy(
            ids_hbm.at[pl.ds((base + CH) * S, CH * S)], idx1, isem)
        fire(idx0, 0, rb0, gsem0)

        def outer(i, carry):
            cbody(2 * i, 0)
            cbody(2 * i + 1, 1)
            return carry

        lax.fori_loop(0, n_chunks // 2, outer, 0)
        drain_out()

    return pool(ids_flat, table)


def _mlp_tc(x_sums, w1, b1, w2, b2):
    batch = x_sums.shape[0]
    bt = 2048

    def mlp_body(x_ref, w1_ref, b1_ref, w2_ref, b2_ref, o_ref):
        xs = x_ref[...] * (1.0 / S)
        h = jnp.dot(xs, w1_ref[...], preferred_element_type=jnp.float32)
        h = jnp.maximum(h + b1_ref[...], 0.0)
        o_ref[...] = (
            jnp.dot(h, w2_ref[...], preferred_element_type=jnp.float32)
            + b2_ref[...]
        )

    return pl.pallas_call(
        mlp_body,
        grid=(batch // bt,),
        in_specs=[
            pl.BlockSpec((bt, D), lambda i: (i, 0)),
            pl.BlockSpec((D, H), lambda i: (0, 0)),
            pl.BlockSpec((1, H), lambda i: (0, 0)),
            pl.BlockSpec((H, C), lambda i: (0, 0)),
            pl.BlockSpec((1, C), lambda i: (0, 0)),
        ],
        out_specs=pl.BlockSpec((bt, C), lambda i: (i, 0)),
        out_shape=jax.ShapeDtypeStruct((batch, C), jnp.float32),
    )(x_sums, w1, b1.reshape(1, H), w2, b2.reshape(1, C))


def kernel(input_ids, emb, W1, b1, W2, b2):
    batch = input_ids.shape[0]
    ids = input_ids.reshape(-1).astype(jnp.int32)
    u = ids % TBW
    ids_flat = ids - u + jnp.where(u < TBW // 2, 2 * u, 2 * u - (TBW - 1))
    table_lin = _transpose_tc(emb.T).reshape(-1, emb.shape[1])
    sums = _pool_sc(ids_flat, table_lin, batch)
    return _mlp_tc(sums, W1, b1, W2, b2)


# speedup vs baseline: 1.9106x; 1.0991x over previous
import functools

import jax
import jax.numpy as jnp
from jax import lax
from jax.experimental import pallas as pl
from jax.experimental.pallas import tpu as pltpu
from jax.experimental.pallas import tpu_sc as plsc

D = 64
S = 200
H = 128
C = 16
L = 16

CH = 16
G = 2
GS = G * S
R = 80
NGD = GS // R
GPC = CH // G


TBW = 8192


def _transpose_tc(emb_t):
    nvoc = emb_t.shape[1]
    BW = TBW
    grid = (nvoc + BW - 1) // BW
    nvoc_pad = grid * BW

    def body(x_ref, o_ref):
        t = x_ref[...].T
        o_ref[...] = jnp.concatenate([t[0 : BW // 2], t[BW // 2 :]], axis=1)

    return pl.pallas_call(
        body,
        grid=(grid,),
        in_specs=[pl.BlockSpec((D, BW), lambda i: (0, i))],
        out_specs=pl.BlockSpec((BW // 2, 2 * D), lambda i: (i, 0)),
        out_shape=jax.ShapeDtypeStruct((nvoc_pad // 2, 2 * D), jnp.float32),
    )(emb_t)


def _pool_sc(ids_flat, table, batch):
    info = plsc.get_sparse_core_info()
    nc, ns = info.num_cores, info.num_subcores
    nw = nc * ns
    b_per_w = batch // nw
    n_chunks = b_per_w // CH

    mesh = plsc.VectorSubcoreMesh(core_axis_name="c", subcore_axis_name="s")

    @functools.partial(
        pl.kernel,
        mesh=mesh,
        out_type=jax.ShapeDtypeStruct((batch, D), jnp.float32),
        compiler_params=pltpu.CompilerParams(use_tc_tiling_on_sc=False),
        scratch_types=[
            pltpu.VMEM((CH * S,), jnp.int32),
            pltpu.VMEM((CH * S,), jnp.int32),
            pltpu.VMEM((GS, D), jnp.float32),
            pltpu.VMEM((GS, D), jnp.float32),
            pltpu.VMEM((CH, D), jnp.float32),
            pltpu.VMEM((CH, D), jnp.float32),
            pltpu.SemaphoreType.DMA,
            pltpu.SemaphoreType.DMA,
            pltpu.SemaphoreType.DMA,
            pltpu.SemaphoreType.DMA,
        ],
    )
    def pool(ids_hbm, table_hbm, out_hbm, idx0, idx1, rb0, rb1, sm0, sm1,
             isem, gsem0, gsem1, osem):
        wid = lax.axis_index("s") * nc + lax.axis_index("c")
        base = wid * b_per_w
        idxs = (idx0, idx1)
        rbs = (rb0, rb1)
        gsems = (gsem0, gsem1)
        sms = (sm0, sm1)

        def fire(idxbuf, gi, rb, sem):
            for c in range(NGD):
                pltpu.async_copy(
                    table_hbm.at[idxbuf.at[pl.ds(gi * GS + c * R, R)]],
                    rb.at[pl.ds(c * R, R)],
                    sem,
                )

        def drain_rows(rb, sem):
            pltpu.make_async_copy(table_hbm.at[pl.ds(0, GS)], rb, sem).wait()

        def drain_idx(idxbuf):
            pltpu.make_async_copy(
                ids_hbm.at[pl.ds(0, CH * S)], idxbuf, isem).wait()

        def drain_out():
            pltpu.make_async_copy(
                sms[0], out_hbm.at[pl.ds(base, CH)], osem).wait()

        def reduce(rb, smbuf, e0):
            def body(j, accs):
                accs = list(accs)
                for u in range(8):
                    r = j * 8 + u
                    for d in range(4):
                        accs[d] = accs[d] + rb[r, pl.ds(d * L, L)]
                        accs[4 + d] = accs[4 + d] + rb[S + r, pl.ds(d * L, L)]
                return tuple(accs)

            z = jnp.zeros((L,), jnp.float32)
            accs = lax.fori_loop(0, S // 8, body, (z,) * 8)
            for d in range(4):
                smbuf[e0, pl.ds(d * L, L)] = accs[d]
                smbuf[e0 + 1, pl.ds(d * L, L)] = accs[4 + d]

        def cbody(ci, p):
            start = base + ci * CH
            for gi in range(GPC):
                par = gi % 2
                if gi < GPC - 1:
                    fire(idxs[p], gi + 1, rbs[1 - par], gsems[1 - par])
                else:
                    @pl.when(ci + 1 < n_chunks)
                    def _():
                        drain_idx(idxs[1 - p])
                        fire(idxs[1 - p], 0, rbs[0], gsems[0])
                drain_rows(rbs[par], gsems[par])
                reduce(rbs[par], sms[p], G * gi)

            @pl.when(ci > 0)
            def _():
                drain_out()
            pltpu.async_copy(sms[p], out_hbm.at[pl.ds(start, CH)], osem)

            @pl.when(ci + 2 < n_chunks)
            def _():
                pltpu.async_copy(
                    ids_hbm.at[pl.ds((start + 2 * CH) * S, CH * S)],
                    idxs[p], isem)

        pltpu.sync_copy(ids_hbm.at[pl.ds(base * S, CH * S)], idx0)
        pltpu.async_copy(
            ids_hbm.at[pl.ds((base + CH) * S, CH * S)], idx1, isem)
        fire(idx0, 0, rb0, gsem0)

        def outer(i, carry):
            cbody(2 * i, 0)
            cbody(2 * i + 1, 1)
            return carry

        lax.fori_loop(0, n_chunks // 2, outer, 0)
        drain_out()

    return pool(ids_flat, table)


def _mlp_tc(x_sums, w1, b1, w2, b2):
    batch = x_sums.shape[0]
    bt = 2048

    def mlp_body(x_ref, w1_ref, b1_ref, w2_ref, b2_ref, o_ref):
        xs = x_ref[...] * (1.0 / S)
        h = jnp.dot(xs, w1_ref[...], preferred_element_type=jnp.float32)
        h = jnp.maximum(h + b1_ref[...], 0.0)
        o_ref[...] = (
            jnp.dot(h, w2_ref[...], preferred_element_type=jnp.float32)
            + b2_ref[...]
        )

    return pl.pallas_call(
        mlp_body,
        grid=(batch // bt,),
        in_specs=[
            pl.BlockSpec((bt, D), lambda i: (i, 0)),
            pl.BlockSpec((D, H), lambda i: (0, 0)),
            pl.BlockSpec((1, H), lambda i: (0, 0)),
            pl.BlockSpec((H, C), lambda i: (0, 0)),
            pl.BlockSpec((1, C), lambda i: (0, 0)),
        ],
        out_specs=pl.BlockSpec((bt, C), lambda i: (i, 0)),
        out_shape=jax.ShapeDtypeStruct((batch, C), jnp.float32),
    )(x_sums, w1, b1.reshape(1, H), w2, b2.reshape(1, C))


def kernel(input_ids, emb, W1, b1, W2, b2):
    batch = input_ids.shape[0]
    ids = input_ids.reshape(-1).astype(jnp.int32)
    u = ids % TBW
    ids_flat = ids - u + jnp.where(u < TBW // 2, 2 * u, 2 * u - (TBW - 1))
    table_lin = _transpose_tc(emb.T).reshape(-1, emb.shape[1])
    sums = _pool_sc(ids_flat, table_lin, batch)
    return _mlp_tc(sums, W1, b1, W2, b2)


# speedup vs baseline: 2.0040x; 1.0489x over previous
import functools

import jax
import jax.numpy as jnp
from jax import lax
from jax.experimental import pallas as pl
from jax.experimental.pallas import tpu as pltpu
from jax.experimental.pallas import tpu_sc as plsc

D = 64
S = 200
H = 128
C = 16
L = 16

CH = 16
G = 2
GS = G * S
R = 80
NGD = GS // R
GPC = CH // G


TBW = 16384


def _transpose_tc(emb_t):
    nvoc = emb_t.shape[1]
    BW = TBW
    grid = (nvoc + BW - 1) // BW
    nvoc_pad = grid * BW

    def body(x_ref, o_ref):
        t = x_ref[...].T
        o_ref[...] = jnp.concatenate([t[0 : BW // 2], t[BW // 2 :]], axis=1)

    return pl.pallas_call(
        body,
        grid=(grid,),
        in_specs=[pl.BlockSpec((D, BW), lambda i: (0, i))],
        out_specs=pl.BlockSpec((BW // 2, 2 * D), lambda i: (i, 0)),
        out_shape=jax.ShapeDtypeStruct((nvoc_pad // 2, 2 * D), jnp.float32),
    )(emb_t)


def _pool_sc(ids_flat, table, batch):
    info = plsc.get_sparse_core_info()
    nc, ns = info.num_cores, info.num_subcores
    nw = nc * ns
    b_per_w = batch // nw
    n_chunks = b_per_w // CH

    mesh = plsc.VectorSubcoreMesh(core_axis_name="c", subcore_axis_name="s")

    @functools.partial(
        pl.kernel,
        mesh=mesh,
        out_type=jax.ShapeDtypeStruct((batch, D), jnp.float32),
        compiler_params=pltpu.CompilerParams(use_tc_tiling_on_sc=False),
        scratch_types=[
            pltpu.VMEM((CH * S,), jnp.int32),
            pltpu.VMEM((CH * S,), jnp.int32),
            pltpu.VMEM((GS, D), jnp.float32),
            pltpu.VMEM((GS, D), jnp.float32),
            pltpu.VMEM((CH, D), jnp.float32),
            pltpu.VMEM((CH, D), jnp.float32),
            pltpu.SemaphoreType.DMA,
            pltpu.SemaphoreType.DMA,
            pltpu.SemaphoreType.DMA,
            pltpu.SemaphoreType.DMA,
        ],
    )
    def pool(ids_hbm, table_hbm, out_hbm, idx0, idx1, rb0, rb1, sm0, sm1,
             isem, gsem0, gsem1, osem):
        wid = lax.axis_index("s") * nc + lax.axis_index("c")
        base = wid * b_per_w
        idxs = (idx0, idx1)
        rbs = (rb0, rb1)
        gsems = (gsem0, gsem1)
        sms = (sm0, sm1)

        def fire(idxbuf, gi, rb, sem):
            for c in range(NGD):
                pltpu.async_copy(
                    table_hbm.at[idxbuf.at[pl.ds(gi * GS + c * R, R)]],
                    rb.at[pl.ds(c * R, R)],
                    sem,
                )

        def drain_rows(rb, sem):
            pltpu.make_async_copy(table_hbm.at[pl.ds(0, GS)], rb, sem).wait()

        def drain_idx(idxbuf):
            pltpu.make_async_copy(
                ids_hbm.at[pl.ds(0, CH * S)], idxbuf, isem).wait()

        def drain_out():
            pltpu.make_async_copy(
                sms[0], out_hbm.at[pl.ds(base, CH)], osem).wait()

        def reduce(rb, smbuf, e0):
            def body(j, accs):
                accs = list(accs)
                for u in range(8):
                    r = j * 8 + u
                    for d in range(4):
                        accs[d] = accs[d] + rb[r, pl.ds(d * L, L)]
                        accs[4 + d] = accs[4 + d] + rb[S + r, pl.ds(d * L, L)]
                return tuple(accs)

            z = jnp.zeros((L,), jnp.float32)
            accs = lax.fori_loop(0, S // 8, body, (z,) * 8)
            for d in range(4):
                smbuf[e0, pl.ds(d * L, L)] = accs[d]
                smbuf[e0 + 1, pl.ds(d * L, L)] = accs[4 + d]

        def cbody(ci, p):
            start = base + ci * CH
            for gi in range(GPC):
                par = gi % 2
                if gi < GPC - 1:
                    fire(idxs[p], gi + 1, rbs[1 - par], gsems[1 - par])
                else:
                    @pl.when(ci + 1 < n_chunks)
                    def _():
                        drain_idx(idxs[1 - p])
                        fire(idxs[1 - p], 0, rbs[0], gsems[0])
                drain_rows(rbs[par], gsems[par])
                reduce(rbs[par], sms[p], G * gi)

            @pl.when(ci > 0)
            def _():
                drain_out()
            pltpu.async_copy(sms[p], out_hbm.at[pl.ds(start, CH)], osem)

            @pl.when(ci + 2 < n_chunks)
            def _():
                pltpu.async_copy(
                    ids_hbm.at[pl.ds((start + 2 * CH) * S, CH * S)],
                    idxs[p], isem)

        pltpu.sync_copy(ids_hbm.at[pl.ds(base * S, CH * S)], idx0)
        pltpu.async_copy(
            ids_hbm.at[pl.ds((base + CH) * S, CH * S)], idx1, isem)
        fire(idx0, 0, rb0, gsem0)

        def outer(i, carry):
            cbody(2 * i, 0)
            cbody(2 * i + 1, 1)
            return carry

        lax.fori_loop(0, n_chunks // 2, outer, 0)
        drain_out()

    return pool(ids_flat, table)


def _mlp_tc(x_sums, w1, b1, w2, b2):
    batch = x_sums.shape[0]
    bt = 2048

    def mlp_body(x_ref, w1_ref, b1_ref, w2_ref, b2_ref, o_ref):
        xs = x_ref[...] * (1.0 / S)
        h = jnp.dot(xs, w1_ref[...], preferred_element_type=jnp.float32)
        h = jnp.maximum(h + b1_ref[...], 0.0)
        o_ref[...] = (
            jnp.dot(h, w2_ref[...], preferred_element_type=jnp.float32)
            + b2_ref[...]
        )

    return pl.pallas_call(
        mlp_body,
        grid=(batch // bt,),
        in_specs=[
            pl.BlockSpec((bt, D), lambda i: (i, 0)),
            pl.BlockSpec((D, H), lambda i: (0, 0)),
            pl.BlockSpec((1, H), lambda i: (0, 0)),
            pl.BlockSpec((H, C), lambda i: (0, 0)),
            pl.BlockSpec((1, C), lambda i: (0, 0)),
        ],
        out_specs=pl.BlockSpec((bt, C), lambda i: (i, 0)),
        out_shape=jax.ShapeDtypeStruct((batch, C), jnp.float32),
    )(x_sums, w1, b1.reshape(1, H), w2, b2.reshape(1, C))


def kernel(input_ids, emb, W1, b1, W2, b2):
    batch = input_ids.shape[0]
    ids = input_ids.reshape(-1).astype(jnp.int32)
    u = ids % TBW
    ids_flat = ids - u + jnp.where(u < TBW // 2, 2 * u, 2 * u - (TBW - 1))
    table_lin = _transpose_tc(emb.T).reshape(-1, emb.shape[1])
    sums = _pool_sc(ids_flat, table_lin, batch)
    return _mlp_tc(sums, W1, b1, W2, b2)


# speedup vs baseline: 2.0494x; 1.0226x over previous
import functools

import jax
import jax.numpy as jnp
from jax import lax
from jax.experimental import pallas as pl
from jax.experimental.pallas import tpu as pltpu
from jax.experimental.pallas import tpu_sc as plsc

D = 64
S = 200
H = 128
C = 16
L = 16

CH = 16
G = 2
GS = G * S
R = 80
NGD = GS // R
GPC = CH // G


TBW = 32768


def _transpose_tc(emb_t):
    nvoc = emb_t.shape[1]
    BW = TBW
    grid = (nvoc + BW - 1) // BW
    nvoc_pad = grid * BW

    def body(x_ref, o_ref):
        t = x_ref[...].T
        o_ref[...] = jnp.concatenate([t[0 : BW // 2], t[BW // 2 :]], axis=1)

    return pl.pallas_call(
        body,
        grid=(grid,),
        in_specs=[pl.BlockSpec((D, BW), lambda i: (0, i))],
        out_specs=pl.BlockSpec((BW // 2, 2 * D), lambda i: (i, 0)),
        out_shape=jax.ShapeDtypeStruct((nvoc_pad // 2, 2 * D), jnp.float32),
    )(emb_t)


def _pool_sc(ids_flat, table, batch):
    info = plsc.get_sparse_core_info()
    nc, ns = info.num_cores, info.num_subcores
    nw = nc * ns
    b_per_w = batch // nw
    n_chunks = b_per_w // CH

    mesh = plsc.VectorSubcoreMesh(core_axis_name="c", subcore_axis_name="s")

    @functools.partial(
        pl.kernel,
        mesh=mesh,
        out_type=jax.ShapeDtypeStruct((batch, D), jnp.float32),
        compiler_params=pltpu.CompilerParams(use_tc_tiling_on_sc=False),
        scratch_types=[
            pltpu.VMEM((CH * S,), jnp.int32),
            pltpu.VMEM((CH * S,), jnp.int32),
            pltpu.VMEM((GS, D), jnp.float32),
            pltpu.VMEM((GS, D), jnp.float32),
            pltpu.VMEM((CH, D), jnp.float32),
            pltpu.VMEM((CH, D), jnp.float32),
            pltpu.SemaphoreType.DMA,
            pltpu.SemaphoreType.DMA,
            pltpu.SemaphoreType.DMA,
            pltpu.SemaphoreType.DMA,
        ],
    )
    def pool(ids_hbm, table_hbm, out_hbm, idx0, idx1, rb0, rb1, sm0, sm1,
             isem, gsem0, gsem1, osem):
        wid = lax.axis_index("s") * nc + lax.axis_index("c")
        base = wid * b_per_w
        idxs = (idx0, idx1)
        rbs = (rb0, rb1)
        gsems = (gsem0, gsem1)
        sms = (sm0, sm1)

        def fire(idxbuf, gi, rb, sem):
            for c in range(NGD):
                pltpu.async_copy(
                    table_hbm.at[idxbuf.at[pl.ds(gi * GS + c * R, R)]],
                    rb.at[pl.ds(c * R, R)],
                    sem,
                )

        def drain_rows(rb, sem):
            pltpu.make_async_copy(table_hbm.at[pl.ds(0, GS)], rb, sem).wait()

        def drain_idx(idxbuf):
            pltpu.make_async_copy(
                ids_hbm.at[pl.ds(0, CH * S)], idxbuf, isem).wait()

        def drain_out():
            pltpu.make_async_copy(
                sms[0], out_hbm.at[pl.ds(base, CH)], osem).wait()

        def reduce(rb, smbuf, e0):
            def body(j, accs):
                accs = list(accs)
                for u in range(8):
                    r = j * 8 + u
                    for d in range(4):
                        accs[d] = accs[d] + rb[r, pl.ds(d * L, L)]
                        accs[4 + d] = accs[4 + d] + rb[S + r, pl.ds(d * L, L)]
                return tuple(accs)

            z = jnp.zeros((L,), jnp.float32)
            accs = lax.fori_loop(0, S // 8, body, (z,) * 8)
            for d in range(4):
                smbuf[e0, pl.ds(d * L, L)] = accs[d]
                smbuf[e0 + 1, pl.ds(d * L, L)] = accs[4 + d]

        def cbody(ci, p):
            start = base + ci * CH
            for gi in range(GPC):
                par = gi % 2
                if gi < GPC - 1:
                    fire(idxs[p], gi + 1, rbs[1 - par], gsems[1 - par])
                else:
                    @pl.when(ci + 1 < n_chunks)
                    def _():
                        drain_idx(idxs[1 - p])
                        fire(idxs[1 - p], 0, rbs[0], gsems[0])
                drain_rows(rbs[par], gsems[par])
                reduce(rbs[par], sms[p], G * gi)

            @pl.when(ci > 0)
            def _():
                drain_out()
            pltpu.async_copy(sms[p], out_hbm.at[pl.ds(start, CH)], osem)

            @pl.when(ci + 2 < n_chunks)
            def _():
                pltpu.async_copy(
                    ids_hbm.at[pl.ds((start + 2 * CH) * S, CH * S)],
                    idxs[p], isem)

        pltpu.sync_copy(ids_hbm.at[pl.ds(base * S, CH * S)], idx0)
        pltpu.async_copy(
            ids_hbm.at[pl.ds((base + CH) * S, CH * S)], idx1, isem)
        fire(idx0, 0, rb0, gsem0)

        def outer(i, carry):
            cbody(2 * i, 0)
            cbody(2 * i + 1, 1)
            return carry

        lax.fori_loop(0, n_chunks // 2, outer, 0)
        drain_out()

    return pool(ids_flat, table)


def _mlp_tc(x_sums, w1, b1, w2, b2):
    batch = x_sums.shape[0]
    bt = 2048

    def mlp_body(x_ref, w1_ref, b1_ref, w2_ref, b2_ref, o_ref):
        xs = x_ref[...] * (1.0 / S)
        h = jnp.dot(xs, w1_ref[...], preferred_element_type=jnp.float32)
        h = jnp.maximum(h + b1_ref[...], 0.0)
        o_ref[...] = (
            jnp.dot(h, w2_ref[...], preferred_element_type=jnp.float32)
            + b2_ref[...]
        )

    return pl.pallas_call(
        mlp_body,
        grid=(batch // bt,),
        in_specs=[
            pl.BlockSpec((bt, D), lambda i: (i, 0)),
            pl.BlockSpec((D, H), lambda i: (0, 0)),
            pl.BlockSpec((1, H), lambda i: (0, 0)),
            pl.BlockSpec((H, C), lambda i: (0, 0)),
            pl.BlockSpec((1, C), lambda i: (0, 0)),
        ],
        out_specs=pl.BlockSpec((bt, C), lambda i: (i, 0)),
        out_shape=jax.ShapeDtypeStruct((batch, C), jnp.float32),
    )(x_sums, w1, b1.reshape(1, H), w2, b2.reshape(1, C))


def kernel(input_ids, emb, W1, b1, W2, b2):
    batch = input_ids.shape[0]
    ids = input_ids.reshape(-1).astype(jnp.int32)
    u = ids % TBW
    ids_flat = ids - u + jnp.where(u < TBW // 2, 2 * u, 2 * u - (TBW - 1))
    table_lin = _transpose_tc(emb.T).reshape(-1, emb.shape[1])
    sums = _pool_sc(ids_flat, table_lin, batch)
    return _mlp_tc(sums, W1, b1, W2, b2)
